# Initial kernel scaffold; baseline (speedup 1.0000x reference)
import jax
import jax.numpy as jnp
from jax.experimental import pallas as pl


def kernel(x, pos_edge_index, neg_edge_index, W_in, b_in, Wpl1, Wpr1, bpr1, Wnl1, Wnr1, bnr1, Wpl2, Wpr2, bpr2, Wnl2, Wnr2, bnr2):
    raise NotImplementedError("write your pallas kernel here")


# speedup vs baseline: 5.1538x; 5.1538x over previous
import functools

import jax
import jax.numpy as jnp
from jax import lax
from jax.experimental import pallas as pl
from jax.experimental.pallas import tpu as pltpu
from jax.experimental.pallas import tpu_sc as plsc

N = 50000
NP = 51200
E = 400000
EPT = 25088
NBATCH = 196
BB = 128
EPAD = 16 * EPT
RPT = NP // 16
WCH = 400
CH = 16
NCH = 8
G = 7
NOUT = NBATCH // (2 * G)

D = 128
HH = 64


def _sc_agg_body(psrc, pdst, nsrc, ndst, feat8, zrow, zcnt, onesb,
                 pos_out, neg_out, cntp_out, cntn_out,
                 dstv, srcb0, srcb1, rows0, rows1, zv, wbv, onesv, cwv,
                 acc, cnt, semg, semi, sems0, sems1):
    c = lax.axis_index("c")
    s = lax.axis_index("s")
    row0 = s * RPT

    pltpu.sync_copy(zrow, zv)
    pltpu.sync_copy(onesb, onesv)

    pl.when(c == 0)(lambda: pltpu.sync_copy(pdst.at[s], dstv))
    pl.when(c == 1)(lambda: pltpu.sync_copy(ndst.at[s], dstv))
    pltpu.sync_copy(zcnt, cwv)
    pltpu.sync_copy(cwv, cnt.at[pl.ds(row0, RPT)])
    plsc.subcore_barrier()

    def cnt_fire(jb, carry):
        pltpu.async_copy(onesv, cnt.at[dstv.at[jb]], semg, add=True)
        return carry
    lax.fori_loop(0, NBATCH, cnt_fire, 0)

    def cnt_drain(jb, carry):
        pltpu.make_async_copy(onesv, cnt.at[dstv.at[jb]], semg).wait()
        return carry
    lax.fori_loop(0, NBATCH, cnt_drain, 0)
    plsc.subcore_barrier()
    pltpu.sync_copy(cnt.at[pl.ds(row0, RPT)], cwv)
    pl.when(c == 0)(lambda: pltpu.sync_copy(cwv, cntp_out.at[pl.ds(row0, RPT)]))
    pl.when(c == 1)(lambda: pltpu.sync_copy(cwv, cntn_out.at[pl.ds(row0, RPT)]))

    def one_pass(chunk, out_ref, src_hs):
        feat = feat8.at[chunk]
        col0 = chunk * CH

        def fire_i(g0, ib):
            pltpu.async_copy(src_hs.at[pl.ds(g0, G)], ib, semi)

        def drain_i(g0, ib):
            pltpu.make_async_copy(src_hs.at[pl.ds(g0, G)], ib, semi).wait()

        def fire_g(g0, ib, buf):
            for t in range(G):
                pltpu.async_copy(feat.at[ib.at[t]], buf.at[t], semg)

        def drain_g(g0, ib, buf):
            for t in range(G):
                pltpu.make_async_copy(feat.at[ib.at[t]], buf.at[t],
                                      semg).wait()

        def fire_s(g0, buf, sem):
            for t in range(G):
                pltpu.async_copy(buf.at[t], acc.at[dstv.at[g0 * G + t]], sem,
                                 add=True)

        def drain_s(g0, buf, sem):
            for t in range(G):
                pltpu.make_async_copy(
                    buf.at[t], acc.at[dstv.at[g0 * G + t]], sem).wait()

        for w in range(RPT // WCH):
            pltpu.sync_copy(zv, acc.at[pl.ds(row0 + w * WCH, WCH)])
        plsc.subcore_barrier()

        pltpu.sync_copy(src_hs.at[pl.ds(0, G)], srcb0)
        fire_g(0, srcb0, rows0)
        fire_i(G, srcb1)

        def outer(i, carry):
            gA = 2 * i
            gB = gA + 1
            drain_g(gA, srcb0, rows0)
            fire_s(gA, rows0, sems0)
            pl.when(i > 0)(lambda: drain_s(gA - 1, rows1, sems1))
            pl.when(i < NOUT - 1)(lambda: fire_i((gA + 2) * G, srcb0))
            drain_i(gB * G, srcb1)
            fire_g(gB, srcb1, rows1)
            drain_g(gB, srcb1, rows1)
            fire_s(gB, rows1, sems1)
            drain_s(gA, rows0, sems0)

            def _next():
                fire_i((gB + 2) * G, srcb1)
                drain_i((gB + 1) * G, srcb0)
                fire_g(gB + 1, srcb0, rows0)
            pl.when(i < NOUT - 1)(_next)
            return carry
        lax.fori_loop(0, NOUT, outer, 0)
        drain_s(2 * NOUT - 1, rows1, sems1)
        plsc.subcore_barrier()

        for w in range(RPT // WCH):
            r0 = row0 + w * WCH
            pltpu.sync_copy(acc.at[pl.ds(r0, WCH)], wbv)
            pltpu.sync_copy(wbv, out_ref.at[pl.ds(r0, WCH), pl.ds(col0, CH)])

    for set_i in range(2):
        src_h = psrc if set_i == 0 else nsrc
        dst_h = pdst if set_i == 0 else ndst
        out_ref = pos_out if set_i == 0 else neg_out
        pltpu.sync_copy(dst_h.at[s], dstv)
        for j in range(4):
            for cc in range(2):
                chunk = 4 * cc + j
                pl.when(c == cc)(
                    functools.partial(one_pass, chunk, out_ref, src_h.at[s]))


def _sc_aggregate(feat8, psrc, pdst, nsrc, ndst, consts):
    zrow, zcnt, onesb = consts
    mesh = plsc.VectorSubcoreMesh(core_axis_name="c", subcore_axis_name="s")
    f32 = jnp.float32
    out_type = (
        jax.ShapeDtypeStruct((NP, D), f32),
        jax.ShapeDtypeStruct((NP, D), f32),
        jax.ShapeDtypeStruct((NP,), f32),
        jax.ShapeDtypeStruct((NP,), f32),
    )
    scratch = [
        pltpu.VMEM((NBATCH, BB), jnp.int32),
        pltpu.VMEM((G, BB), jnp.int32),
        pltpu.VMEM((G, BB), jnp.int32),
        pltpu.VMEM((G, BB, CH), f32),
        pltpu.VMEM((G, BB, CH), f32),
        pltpu.VMEM((WCH, CH), f32),
        pltpu.VMEM((WCH, CH), f32),
        pltpu.VMEM((BB,), f32),
        pltpu.VMEM((RPT,), f32),
        pltpu.VMEM_SHARED((NP, CH), f32),
        pltpu.VMEM_SHARED((NP,), f32),
        pltpu.SemaphoreType.DMA,
        pltpu.SemaphoreType.DMA,
        pltpu.SemaphoreType.DMA,
        pltpu.SemaphoreType.DMA,
    ]
    call = pl.kernel(
        _sc_agg_body, out_type=out_type, mesh=mesh, scratch_types=scratch,
        compiler_params=pltpu.CompilerParams(use_tc_tiling_on_sc=False))
    return call(psrc, pdst, nsrc, ndst, feat8, zrow, zcnt, onesb)


BN = 512


def _tc1_body(x, w, b, h_ref, h8_ref):
    h = jnp.dot(x[...], w[...], preferred_element_type=jnp.float32) + b[...]
    h = jnp.maximum(h, 0.0)
    h_ref[...] = h
    for k in range(NCH):
        h8_ref[k] = h[:, k * CH:(k + 1) * CH]


def _tc1(x, W_in, b_in):
    grid = (NP // BN,)
    f32 = jnp.float32
    return pl.pallas_call(
        _tc1_body,
        grid=grid,
        in_specs=[
            pl.BlockSpec((BN, D), lambda i: (i, 0)),
            pl.BlockSpec((D, D), lambda i: (0, 0)),
            pl.BlockSpec((1, D), lambda i: (0, 0)),
        ],
        out_specs=[
            pl.BlockSpec((BN, D), lambda i: (i, 0)),
            pl.BlockSpec((NCH, BN, CH), lambda i: (0, i, 0)),
        ],
        out_shape=[
            jax.ShapeDtypeStruct((NP, D), f32),
            jax.ShapeDtypeStruct((NCH, NP, CH), f32),
        ],
    )(x, W_in, b_in.reshape(1, D))


def _tc2_body(h, ps, ns, cp, cn, wpl, wpr, bpr, wnl, wnr, bnr, z_ref, z8_ref):
    rp = 1.0 / jnp.maximum(cp[...], 1.0)
    rn = 1.0 / jnp.maximum(cn[...], 1.0)
    p = ps[...] * rp
    ng = ns[...] * rn
    hb = h[...]
    out_pos = (jnp.dot(p, wpl[...], preferred_element_type=jnp.float32)
               + jnp.dot(hb, wpr[...], preferred_element_type=jnp.float32)
               + bpr[...])
    out_neg = (jnp.dot(ng, wnl[...], preferred_element_type=jnp.float32)
               + jnp.dot(hb, wnr[...], preferred_element_type=jnp.float32)
               + bnr[...])
    z = jnp.maximum(jnp.concatenate([out_pos, out_neg], axis=1), 0.0)
    z_ref[...] = z
    for k in range(NCH):
        z8_ref[k] = z[:, k * CH:(k + 1) * CH]


def _tc2(h, ps, ns, cp, cn, Wpl1, Wpr1, bpr1, Wnl1, Wnr1, bnr1):
    grid = (NP // BN,)
    f32 = jnp.float32
    full = lambda r, c_: pl.BlockSpec((r, c_), lambda i: (0, 0))
    return pl.pallas_call(
        _tc2_body,
        grid=grid,
        in_specs=[
            pl.BlockSpec((BN, D), lambda i: (i, 0)),
            pl.BlockSpec((BN, D), lambda i: (i, 0)),
            pl.BlockSpec((BN, D), lambda i: (i, 0)),
            pl.BlockSpec((BN, 1), lambda i: (i, 0)),
            pl.BlockSpec((BN, 1), lambda i: (i, 0)),
            full(D, HH), full(D, HH), full(1, HH),
            full(D, HH), full(D, HH), full(1, HH),
        ],
        out_specs=[
            pl.BlockSpec((BN, D), lambda i: (i, 0)),
            pl.BlockSpec((NCH, BN, CH), lambda i: (0, i, 0)),
        ],
        out_shape=[
            jax.ShapeDtypeStruct((NP, D), f32),
            jax.ShapeDtypeStruct((NCH, NP, CH), f32),
        ],
    )(h, ps, ns, cp, cn, Wpl1, Wpr1, bpr1.reshape(1, HH),
      Wnl1, Wnr1, bnr1.reshape(1, HH))


def _tc3_body(z, zps, zns, cp, cn, wpl, wpr, bpr, wnl, wnr, bnr, out_ref):
    rp = 1.0 / jnp.maximum(cp[...], 1.0)
    rn = 1.0 / jnp.maximum(cn[...], 1.0)
    zb = z[...]
    zp = zb[:, :HH]
    zn = zb[:, HH:]
    psum = zps[...] * rp
    nsum = zns[...] * rn
    op = jnp.concatenate([psum[:, :HH], nsum[:, HH:]], axis=1)
    on = jnp.concatenate([psum[:, HH:], nsum[:, :HH]], axis=1)
    out_pos = (jnp.dot(op, wpl[...], preferred_element_type=jnp.float32)
               + jnp.dot(zp, wpr[...], preferred_element_type=jnp.float32)
               + bpr[...])
    out_neg = (jnp.dot(on, wnl[...], preferred_element_type=jnp.float32)
               + jnp.dot(zn, wnr[...], preferred_element_type=jnp.float32)
               + bnr[...])
    out_ref[...] = jnp.maximum(jnp.concatenate([out_pos, out_neg], axis=1), 0.0)


def _tc3(z, zps, zns, cp, cn, Wpl2, Wpr2, bpr2, Wnl2, Wnr2, bnr2):
    grid = (NP // BN,)
    f32 = jnp.float32
    full = lambda r, c_: pl.BlockSpec((r, c_), lambda i: (0, 0))
    return pl.pallas_call(
        _tc3_body,
        grid=grid,
        in_specs=[
            pl.BlockSpec((BN, D), lambda i: (i, 0)),
            pl.BlockSpec((BN, D), lambda i: (i, 0)),
            pl.BlockSpec((BN, D), lambda i: (i, 0)),
            pl.BlockSpec((BN, 1), lambda i: (i, 0)),
            pl.BlockSpec((BN, 1), lambda i: (i, 0)),
            full(2 * HH, HH), full(HH, HH), full(1, HH),
            full(2 * HH, HH), full(HH, HH), full(1, HH),
        ],
        out_specs=pl.BlockSpec((BN, D), lambda i: (i, 0)),
        out_shape=jax.ShapeDtypeStruct((NP, D), f32),
    )(z, zps, zns, cp, cn, Wpl2, Wpr2, bpr2.reshape(1, HH),
      Wnl2, Wnr2, bnr2.reshape(1, HH))


def _prep_edges(ei):
    src = jnp.concatenate(
        [ei[0], jnp.zeros((EPAD - E,), jnp.int32)]).reshape(16, NBATCH, BB)
    dst = jnp.concatenate(
        [ei[1], jnp.full((EPAD - E,), NP - 2, jnp.int32)]).reshape(16, NBATCH, BB)
    return src, dst


def kernel(x, pos_edge_index, neg_edge_index, W_in, b_in, Wpl1, Wpr1, bpr1,
           Wnl1, Wnr1, bnr1, Wpl2, Wpr2, bpr2, Wnl2, Wnr2, bnr2):
    f32 = jnp.float32
    xp = jnp.pad(x, ((0, NP - N), (0, 0)))
    psrc, pdst = _prep_edges(pos_edge_index)
    nsrc, ndst = _prep_edges(neg_edge_index)
    consts = (jnp.zeros((WCH, CH), f32), jnp.zeros((RPT,), f32),
              jnp.ones((BB,), f32))

    h, h8 = _tc1(xp, W_in, b_in)
    ps, ns, cp, cn = _sc_aggregate(h8, psrc, pdst, nsrc, ndst, consts)
    cp2 = cp.reshape(NP, 1)
    cn2 = cn.reshape(NP, 1)
    z, z8 = _tc2(h, ps, ns, cp2, cn2, Wpl1, Wpr1, bpr1, Wnl1, Wnr1, bnr1)
    zps, zns, _, _ = _sc_aggregate(z8, psrc, pdst, nsrc, ndst, consts)
    z2 = _tc3(z, zps, zns, cp2, cn2, Wpl2, Wpr2, bpr2, Wnl2, Wnr2, bnr2)
    return z2[:N]
